# baseline (speedup 1.0000x reference)
import jax
import jax.numpy as jnp
from jax import lax
from jax.experimental import pallas as pl
from jax.experimental.pallas import tpu as pltpu
from jax.experimental.pallas import tpu_sc as plsc

B = 16384
D = 64
L = 16
NW = 32
BPW = B // NW
NBUF = 12
OCH = 64


def _body(tabT, ids, t, w, bias, outv, idx_v, t_v, o_row, w_v, b1_v, bufs, sems):
    wid = lax.axis_index("s") * 2 + lax.axis_index("c")
    base = wid * BPW
    pltpu.sync_copy(ids.at[pl.ds(base, BPW)], idx_v)
    pltpu.sync_copy(t.at[pl.ds(base, BPW)], t_v)
    pltpu.sync_copy(w, w_v)
    pltpu.sync_copy(bias, b1_v)

    w_c = [w_v[pl.ds(c * L, L)] for c in range(4)]
    b1_c = [b1_v[pl.ds(c * L, L)] + 1.0 for c in range(4)]
    iota = lax.iota(jnp.int32, L)
    rows_c = [c * L + iota for c in range(4)]

    def issue(chunk, off, slot):
        v = idx_v[pl.ds(chunk * L, L)]
        i = v[off]
        col0 = pl.multiple_of((i >> 7) << 7, 128)
        pltpu.async_copy(tabT.at[:, pl.ds(col0, 128)], bufs[slot], sems[slot])

    for j0 in range(NBUF):
        issue(0, j0, j0)

    def blk(n, carry):
        v = idx_v[pl.ds(n * L, L)]
        t_blk = t_v[pl.ds(n * L, L)]
        for j in range(L):
            slot = j % NBUF
            pltpu.make_async_copy(tabT.at[:, pl.ds(0, 128)], bufs[slot],
                                  sems[slot]).wait()
            lane = v[j] & 127
            cols = jnp.broadcast_to(lane, (L,))
            t_k = t_blk[j]
            orow = (n % 4) * L + j
            for c in range(4):
                g = plsc.load_gather(bufs[slot], [rows_c[c], cols])
                o_row[orow, pl.ds(c * L, L)] = g * (t_k * w_c[c] + b1_c[c])
            nxt = n * L + j + NBUF
            @pl.when(nxt < BPW)
            def _():
                issue((n * L + j + NBUF) // L, (j + NBUF) % L, slot)
        @pl.when(n % 4 == 3)
        def _():
            pltpu.sync_copy(o_row, outv.at[pl.ds(base + (n // 4) * OCH, OCH)])
        return carry

    lax.fori_loop(0, BPW // L, blk, 0)


@jax.jit
def _tpe(tabT, ids, t, w, bias):
    mesh = plsc.VectorSubcoreMesh(core_axis_name="c", subcore_axis_name="s")
    return pl.kernel(
        _body,
        out_type=jax.ShapeDtypeStruct((B, D), jnp.float32),
        mesh=mesh,
        scratch_types=[
            pltpu.VMEM((BPW,), jnp.int32),
            pltpu.VMEM((BPW,), jnp.float32),
            pltpu.VMEM((OCH, D), jnp.float32),
            pltpu.VMEM((D,), jnp.float32),
            pltpu.VMEM((D,), jnp.float32),
            [pltpu.VMEM((D, 128), jnp.float32) for _ in range(NBUF)],
            [pltpu.SemaphoreType.DMA for _ in range(NBUF)],
        ],
        compiler_params=pltpu.CompilerParams(needs_layout_passes=False),
    )(tabT, ids, t, w, bias)


def kernel(node_memories, node_ids, node_time_intervals, W, b):
    tabT = jnp.swapaxes(node_memories, 0, 1)
    return _tpe(tabT, node_ids.astype(jnp.int32), node_time_intervals, W, b)


# speedup vs baseline: 1.0517x; 1.0517x over previous
import jax
import jax.numpy as jnp
from jax import lax
from jax.experimental import pallas as pl
from jax.experimental.pallas import tpu as pltpu
from jax.experimental.pallas import tpu_sc as plsc

B = 16384
D = 64
L = 16
NW = 32
BPW = B // NW
NBUF = 10
OCH = 128


def _body(tabT, ids, t, w, bias, outv, idx_v, t_v, o_row, w_v, b1_v, bufs, sems):
    wid = lax.axis_index("s") * 2 + lax.axis_index("c")
    base = wid * BPW
    pltpu.sync_copy(ids.at[pl.ds(base, BPW)], idx_v)
    pltpu.sync_copy(t.at[pl.ds(base, BPW)], t_v)
    pltpu.sync_copy(w, w_v)
    pltpu.sync_copy(bias, b1_v)

    w_c = [w_v[pl.ds(c * L, L)] for c in range(4)]
    b1_c = [b1_v[pl.ds(c * L, L)] + 1.0 for c in range(4)]
    iota = lax.iota(jnp.int32, L)
    rows_c = [c * L + iota for c in range(4)]

    def issue(chunk, off, slot):
        v = idx_v[pl.ds(chunk * L, L)]
        i = v[off]
        col0 = pl.multiple_of((i >> 7) << 7, 128)
        pltpu.async_copy(tabT.at[:, pl.ds(col0, 128)], bufs[slot], sems[slot])

    for j0 in range(NBUF):
        issue(0, j0, j0)

    def blk(n, carry):
        v = idx_v[pl.ds(n * L, L)]
        t_blk = t_v[pl.ds(n * L, L)]
        for j in range(L):
            slot = j % NBUF
            pltpu.make_async_copy(tabT.at[:, pl.ds(0, 128)], bufs[slot],
                                  sems[slot]).wait()
            lane = v[j] & 127
            cols = jnp.broadcast_to(lane, (L,))
            t_k = t_blk[j]
            orow = (n % 8) * L + j
            for c in range(4):
                g = plsc.load_gather(bufs[slot], [rows_c[c], cols])
                o_row[orow, pl.ds(c * L, L)] = g * (t_k * w_c[c] + b1_c[c])
            nxt = n * L + j + NBUF
            @pl.when(nxt < BPW)
            def _():
                issue((n * L + j + NBUF) // L, (j + NBUF) % L, slot)
        @pl.when(n % 8 == 7)
        def _():
            pltpu.sync_copy(o_row, outv.at[pl.ds(base + (n // 8) * OCH, OCH)])
        return carry

    lax.fori_loop(0, BPW // L, blk, 0)


@jax.jit
def _tpe(tabT, ids, t, w, bias):
    mesh = plsc.VectorSubcoreMesh(core_axis_name="c", subcore_axis_name="s")
    return pl.kernel(
        _body,
        out_type=jax.ShapeDtypeStruct((B, D), jnp.float32),
        mesh=mesh,
        scratch_types=[
            pltpu.VMEM((BPW,), jnp.int32),
            pltpu.VMEM((BPW,), jnp.float32),
            pltpu.VMEM((OCH, D), jnp.float32),
            pltpu.VMEM((D,), jnp.float32),
            pltpu.VMEM((D,), jnp.float32),
            [pltpu.VMEM((D, 128), jnp.float32) for _ in range(NBUF)],
            [pltpu.SemaphoreType.DMA for _ in range(NBUF)],
        ],
        compiler_params=pltpu.CompilerParams(needs_layout_passes=False),
    )(tabT, ids, t, w, bias)


def kernel(node_memories, node_ids, node_time_intervals, W, b):
    tabT = jnp.swapaxes(node_memories, 0, 1)
    return _tpe(tabT, node_ids.astype(jnp.int32), node_time_intervals, W, b)


# speedup vs baseline: 1.0591x; 1.0070x over previous
import jax
import jax.numpy as jnp
from jax import lax
from jax.experimental import pallas as pl
from jax.experimental.pallas import tpu as pltpu
from jax.experimental.pallas import tpu_sc as plsc

B = 16384
D = 64
L = 16
NW = 32
BPW = B // NW
NBUF = 8
OCH = 128


def _body(tabT, ids, t, w, bias, outv, idx_v, t_v, o_row, w_v, b1_v, bufs, sems):
    wid = lax.axis_index("s") * 2 + lax.axis_index("c")
    base = wid * BPW
    pltpu.sync_copy(ids.at[pl.ds(base, BPW)], idx_v)
    pltpu.sync_copy(t.at[pl.ds(base, BPW)], t_v)
    pltpu.sync_copy(w, w_v)
    pltpu.sync_copy(bias, b1_v)

    w_c = [w_v[pl.ds(c * L, L)] for c in range(4)]
    b1_c = [b1_v[pl.ds(c * L, L)] + 1.0 for c in range(4)]
    iota = lax.iota(jnp.int32, L)
    rows_c = [c * L + iota for c in range(4)]

    def issue_from(cvec, off, slot):
        col0 = pl.multiple_of(cvec[off], 128)
        pltpu.async_copy(tabT.at[:, pl.ds(col0, 128)], bufs[slot], sems[slot])

    v0 = idx_v[pl.ds(0, L)]
    c0vec = (v0 >> 7) << 7
    for j0 in range(NBUF):
        issue_from(c0vec, j0, j0)

    def blk(n, carry):
        v = idx_v[pl.ds(n * L, L)]
        t_blk = t_v[pl.ds(n * L, L)]
        lane_vec = v & 127
        col_vec = (v >> 7) << 7
        nn = jnp.minimum(n + 1, BPW // L - 1)
        vn = idx_v[pl.ds(nn * L, L)]
        coln_vec = (vn >> 7) << 7
        for j in range(L):
            slot = j % NBUF
            pltpu.make_async_copy(tabT.at[:, pl.ds(0, 128)], bufs[slot],
                                  sems[slot]).wait()
            cols = jnp.broadcast_to(lane_vec[j], (L,))
            t_k = t_blk[j]
            orow = (n % 8) * L + j
            for c in range(4):
                g = plsc.load_gather(bufs[slot], [rows_c[c], cols])
                o_row[orow, pl.ds(c * L, L)] = g * (t_k * w_c[c] + b1_c[c])
            if j < L - NBUF:
                issue_from(col_vec, j + NBUF, slot)
            else:
                @pl.when(n + 1 < BPW // L)
                def _():
                    issue_from(coln_vec, (j + NBUF) % L, slot)
        @pl.when(n % 8 == 7)
        def _():
            pltpu.sync_copy(o_row, outv.at[pl.ds(base + (n // 8) * OCH, OCH)])
        return carry

    lax.fori_loop(0, BPW // L, blk, 0)


@jax.jit
def _tpe(tabT, ids, t, w, bias):
    mesh = plsc.VectorSubcoreMesh(core_axis_name="c", subcore_axis_name="s")
    return pl.kernel(
        _body,
        out_type=jax.ShapeDtypeStruct((B, D), jnp.float32),
        mesh=mesh,
        scratch_types=[
            pltpu.VMEM((BPW,), jnp.int32),
            pltpu.VMEM((BPW,), jnp.float32),
            pltpu.VMEM((OCH, D), jnp.float32),
            pltpu.VMEM((D,), jnp.float32),
            pltpu.VMEM((D,), jnp.float32),
            [pltpu.VMEM((D, 128), jnp.float32) for _ in range(NBUF)],
            [pltpu.SemaphoreType.DMA for _ in range(NBUF)],
        ],
        compiler_params=pltpu.CompilerParams(needs_layout_passes=False),
    )(tabT, ids, t, w, bias)


def kernel(node_memories, node_ids, node_time_intervals, W, b):
    tabT = jnp.swapaxes(node_memories, 0, 1)
    return _tpe(tabT, node_ids.astype(jnp.int32), node_time_intervals, W, b)
